# baseline (speedup 1.0000x reference)
import functools

import jax
import jax.numpy as jnp
from jax import lax
from jax.experimental import pallas as pl
from jax.experimental.pallas import tpu as pltpu
from jax.experimental.pallas import tpu_sc as plsc

_B, _H, _D = 2048, 512, 32
_NW = 32
_SPW = _B // _NW


def _sqrt16(x):
    i = plsc.bitcast(x, jnp.int32)
    y = plsc.bitcast(jax.lax.shift_right_logical(i, 1) + 0x1fbd1df5,
                     jnp.float32)
    for _ in range(3):
        y = 0.5 * (y + x / y)
    return y


def _sc_body(pred_hbm, targ_hbm, out_hbm,
             ib1, ib0, g1, g0, gp, acc_v, sems):
    c = lax.axis_index("c")
    s = lax.axis_index("s")
    wid = s * 2 + c
    base = wid * _SPW

    lane = lax.iota(jnp.int32, 16)
    ngrp = _SPW // 16
    for g in range(ngrp):
        rows = (base + g * 16 + lane) * _D
        ib0[g, :] = rows
        ib1[g, :] = rows + 1

    h1 = []
    h0 = []
    hp = []
    for g in range(ngrp):
        sl = pl.ds(g * 16, 16)
        h1.append(pltpu.async_copy(targ_hbm.at[ib1.at[g]], g1.at[sl, :],
                                   sems.at[0, g]))
        h0.append(pltpu.async_copy(targ_hbm.at[ib0.at[g]], g0.at[sl, :],
                                   sems.at[1, g]))
        hp.append(pltpu.async_copy(pred_hbm.at[ib0.at[g]], gp.at[sl, :],
                                   sems.at[2, g]))

    acc = jnp.zeros((16,), jnp.float32)
    for g in range(ngrp):
        h1[g].wait()
        h0[g].wait()
        hp[g].wait()

        def sbody(kk, carry, g=g):
            tvec, t0v, p0v = carry
            j = g * 16 + kk

            def cbody(ci, cnt, j=j):
                c0 = pl.multiple_of(ci * 128, 128)
                x = (g1[j, pl.ds(c0, 16)] != 0.0).astype(jnp.float32)
                for u in range(1, 8):
                    cu = pl.multiple_of(c0 + u * 16, 16)
                    x = x + (g1[j, pl.ds(cu, 16)] != 0.0).astype(jnp.float32)
                return cnt + x

            cnt = lax.fori_loop(0, _H // 128, cbody,
                                jnp.zeros((16,), jnp.float32))
            t = jnp.sum(cnt)
            safe = jnp.maximum(t.astype(jnp.int32) - 1, 0)

            sub = (lane == safe % 16).astype(jnp.float32)
            co16 = pl.multiple_of((safe // 16) * 16, 16)
            t0 = jnp.sum(g0[j, pl.ds(co16, 16)] * sub)
            p0 = jnp.sum(gp[j, pl.ds(co16, 16)] * sub)

            sel = lane == kk
            tvec = jnp.where(sel, jnp.full((16,), t, jnp.float32), tvec)
            t0v = jnp.where(sel, jnp.full((16,), t0, jnp.float32), t0v)
            p0v = jnp.where(sel, jnp.full((16,), p0, jnp.float32), p0v)
            return (tvec, t0v, p0v)

        zeros16 = jnp.zeros((16,), jnp.float32)
        tvec, t0v, p0v = lax.fori_loop(0, 16, sbody,
                                       (zeros16, zeros16, zeros16))

        xn = tvec * (1.0 / (_H - 1))
        w = 1.0 + 0.7 * (xn * xn) * _sqrt16(xn)
        coeff = jnp.where(tvec >= 1.0, w * (_D / _B),
                          jnp.zeros((16,), jnp.float32))
        d = p0v - t0v
        acc = acc + coeff * d * d

    acc_v[pl.ds(0, 16)] = acc
    pltpu.sync_copy(acc_v, out_hbm.at[pl.ds(wid * 128, 128)])


def kernel(pred, targ, weights):
    predT = jnp.transpose(pred, (0, 2, 1)).reshape(_B * _D, _H)
    targT = jnp.transpose(targ, (0, 2, 1)).reshape(_B * _D, _H)

    mesh = plsc.VectorSubcoreMesh(core_axis_name="c", subcore_axis_name="s")
    run = functools.partial(
        pl.kernel,
        mesh=mesh,
        compiler_params=pltpu.CompilerParams(needs_layout_passes=False),
        out_type=jax.ShapeDtypeStruct((_NW * 128,), jnp.float32),
        scratch_types=[
            pltpu.VMEM((_SPW // 16, 16), jnp.int32),
            pltpu.VMEM((_SPW // 16, 16), jnp.int32),
            pltpu.VMEM((_SPW, _H), jnp.float32),
            pltpu.VMEM((_SPW, _H), jnp.float32),
            pltpu.VMEM((_SPW, _H), jnp.float32),
            pltpu.VMEM((128,), jnp.float32),
            pltpu.SemaphoreType.DMA((3, _SPW // 16)),
        ],
    )(_sc_body)

    flat = run(predT, targT)
    partials = flat.reshape(_NW, 128)[:, :16]
    loss = jnp.sum(partials)
    return (loss, {"a0_loss": loss})


# speedup vs baseline: 1.0961x; 1.0961x over previous
import functools

import jax
import jax.numpy as jnp
from jax import lax
from jax.experimental import pallas as pl
from jax.experimental.pallas import tpu as pltpu
from jax.experimental.pallas import tpu_sc as plsc

_B, _H, _D = 2048, 512, 32
_NW = 32
_SPW = _B // _NW


def _sqrt16(x):
    i = plsc.bitcast(x, jnp.int32)
    y = plsc.bitcast(jax.lax.shift_right_logical(i, 1) + 0x1fbd1df5,
                     jnp.float32)
    for _ in range(3):
        y = 0.5 * (y + x / y)
    return y


def _sc_body(pred_hbm, targ_hbm, out_hbm,
             ib1, ib0, g1, g0, gp, acc_v, sems):
    c = lax.axis_index("c")
    s = lax.axis_index("s")
    wid = s * 2 + c
    base = wid * _SPW

    lane = lax.iota(jnp.int32, 16)
    ngrp = _SPW // 16
    for g in range(ngrp):
        rows = (base + g * 16 + lane) * _D
        ib0[g, :] = rows
        ib1[g, :] = rows + 1

    h1 = []
    h0 = []
    hp = []
    for g in range(ngrp):
        sl = pl.ds(g * 16, 16)
        h1.append(pltpu.async_copy(targ_hbm.at[ib1.at[g]], g1.at[sl, :],
                                   sems.at[0, g]))
        h0.append(pltpu.async_copy(targ_hbm.at[ib0.at[g]], g0.at[sl, :],
                                   sems.at[1, g]))
        hp.append(pltpu.async_copy(pred_hbm.at[ib0.at[g]], gp.at[sl, :],
                                   sems.at[2, g]))

    acc = jnp.zeros((16,), jnp.float32)
    for g in range(ngrp):
        tvec = jnp.zeros((16,), jnp.float32)
        t0v = jnp.zeros((16,), jnp.float32)
        p0v = jnp.zeros((16,), jnp.float32)
        h1[g].wait()
        for k in range(16):
            j = g * 16 + k

            def cbody(ci, cnt, j=j):
                c0 = pl.multiple_of(ci * 128, 128)
                x = (g1[j, pl.ds(c0, 16)] != 0.0).astype(jnp.float32)
                for u in range(1, 8):
                    cu = pl.multiple_of(c0 + u * 16, 16)
                    x = x + (g1[j, pl.ds(cu, 16)] != 0.0).astype(jnp.float32)
                return cnt + x

            cnt = lax.fori_loop(0, _H // 128, cbody,
                                jnp.zeros((16,), jnp.float32))
            t = jnp.sum(cnt)
            safe = jnp.maximum(t.astype(jnp.int32) - 1, 0)

            if k == 0:
                h0[g].wait()
                hp[g].wait()
            sub = (lane == safe % 16).astype(jnp.float32)
            co16 = pl.multiple_of((safe // 16) * 16, 16)
            t0 = jnp.sum(g0[j, pl.ds(co16, 16)] * sub)
            p0 = jnp.sum(gp[j, pl.ds(co16, 16)] * sub)

            sel = lane == k
            tvec = jnp.where(sel, jnp.full((16,), t, jnp.float32), tvec)
            t0v = jnp.where(sel, jnp.full((16,), t0, jnp.float32), t0v)
            p0v = jnp.where(sel, jnp.full((16,), p0, jnp.float32), p0v)

        xn = tvec * (1.0 / (_H - 1))
        w = 1.0 + 0.7 * (xn * xn) * _sqrt16(xn)
        coeff = jnp.where(tvec >= 1.0, w * (_D / _B),
                          jnp.zeros((16,), jnp.float32))
        d = p0v - t0v
        acc = acc + coeff * d * d

    acc_v[pl.ds(0, 16)] = acc
    pltpu.sync_copy(acc_v, out_hbm.at[pl.ds(wid * 128, 128)])


def kernel(pred, targ, weights):
    predT = jnp.transpose(pred, (0, 2, 1)).reshape(_B * _D, _H)
    targT = jnp.transpose(targ, (0, 2, 1)).reshape(_B * _D, _H)

    mesh = plsc.VectorSubcoreMesh(core_axis_name="c", subcore_axis_name="s")
    run = functools.partial(
        pl.kernel,
        mesh=mesh,
        compiler_params=pltpu.CompilerParams(needs_layout_passes=False),
        out_type=jax.ShapeDtypeStruct((_NW * 128,), jnp.float32),
        scratch_types=[
            pltpu.VMEM((_SPW // 16, 16), jnp.int32),
            pltpu.VMEM((_SPW // 16, 16), jnp.int32),
            pltpu.VMEM((_SPW, _H), jnp.float32),
            pltpu.VMEM((_SPW, _H), jnp.float32),
            pltpu.VMEM((_SPW, _H), jnp.float32),
            pltpu.VMEM((128,), jnp.float32),
            pltpu.SemaphoreType.DMA((3, _SPW // 16)),
        ],
    )(_sc_body)

    flat = run(predT, targT)
    partials = flat.reshape(_NW, 128)[:, :16]
    loss = jnp.sum(partials)
    return (loss, {"a0_loss": loss})


# speedup vs baseline: 1.1039x; 1.0071x over previous
import functools

import jax
import jax.numpy as jnp
from jax import lax
from jax.experimental import pallas as pl
from jax.experimental.pallas import tpu as pltpu
from jax.experimental.pallas import tpu_sc as plsc

_B, _H, _D = 2048, 512, 32
_NW = 32
_SPW = _B // _NW


def _sqrt16(x):
    i = plsc.bitcast(x, jnp.int32)
    y = plsc.bitcast(jax.lax.shift_right_logical(i, 1) + 0x1fbd1df5,
                     jnp.float32)
    for _ in range(3):
        y = 0.5 * (y + x / y)
    return y


def _sc_body(pred_hbm, targ_hbm, out_hbm,
             g1, g0, gp, acc_v, sems):
    c = lax.axis_index("c")
    s = lax.axis_index("s")
    wid = s * 2 + c
    base = wid * _SPW

    lane = lax.iota(jnp.int32, 16)
    ngrp = _SPW // 16
    h1 = []
    h0 = []
    hp = []
    for g in range(ngrp):
        rows = (base + g * 16 + lane) * _D
        sl = pl.ds(g * 16, 16)
        h1.append(pltpu.async_copy(targ_hbm.at[rows + 1], g1.at[sl, :],
                                   sems.at[0, g]))
        h0.append(pltpu.async_copy(targ_hbm.at[rows], g0.at[sl, :],
                                   sems.at[1, g]))
        hp.append(pltpu.async_copy(pred_hbm.at[rows], gp.at[sl, :],
                                   sems.at[2, g]))

    acc = jnp.zeros((16,), jnp.float32)
    for g in range(ngrp):
        tvec = jnp.zeros((16,), jnp.float32)
        t0v = jnp.zeros((16,), jnp.float32)
        p0v = jnp.zeros((16,), jnp.float32)
        h1[g].wait()
        for k in range(16):
            j = g * 16 + k

            def cbody(ci, cnt, j=j):
                c0 = pl.multiple_of(ci * 128, 128)
                x = (g1[j, pl.ds(c0, 16)] != 0.0).astype(jnp.float32)
                for u in range(1, 8):
                    cu = pl.multiple_of(c0 + u * 16, 16)
                    x = x + (g1[j, pl.ds(cu, 16)] != 0.0).astype(jnp.float32)
                return cnt + x

            cnt = lax.fori_loop(0, _H // 128, cbody,
                                jnp.zeros((16,), jnp.float32))
            t = jnp.sum(cnt)
            safe = jnp.maximum(t.astype(jnp.int32) - 1, 0)

            if k == 0:
                h0[g].wait()
                hp[g].wait()
            sub = (lane == safe % 16).astype(jnp.float32)
            co16 = pl.multiple_of((safe // 16) * 16, 16)
            t0 = jnp.sum(g0[j, pl.ds(co16, 16)] * sub)
            p0 = jnp.sum(gp[j, pl.ds(co16, 16)] * sub)

            sel = lane == k
            tvec = jnp.where(sel, jnp.full((16,), t, jnp.float32), tvec)
            t0v = jnp.where(sel, jnp.full((16,), t0, jnp.float32), t0v)
            p0v = jnp.where(sel, jnp.full((16,), p0, jnp.float32), p0v)

        xn = tvec * (1.0 / (_H - 1))
        w = 1.0 + 0.7 * (xn * xn) * _sqrt16(xn)
        coeff = jnp.where(tvec >= 1.0, w * (_D / _B),
                          jnp.zeros((16,), jnp.float32))
        d = p0v - t0v
        acc = acc + coeff * d * d

    acc_v[pl.ds(0, 16)] = acc
    pltpu.sync_copy(acc_v, out_hbm.at[pl.ds(wid * 128, 128)])


def kernel(pred, targ, weights):
    predT = jnp.transpose(pred, (0, 2, 1)).reshape(_B * _D, _H)
    targT = jnp.transpose(targ, (0, 2, 1)).reshape(_B * _D, _H)

    mesh = plsc.VectorSubcoreMesh(core_axis_name="c", subcore_axis_name="s")
    run = functools.partial(
        pl.kernel,
        mesh=mesh,
        compiler_params=pltpu.CompilerParams(needs_layout_passes=False),
        out_type=jax.ShapeDtypeStruct((_NW * 128,), jnp.float32),
        scratch_types=[
            pltpu.VMEM((_SPW, _H), jnp.float32),
            pltpu.VMEM((_SPW, _H), jnp.float32),
            pltpu.VMEM((_SPW, _H), jnp.float32),
            pltpu.VMEM((128,), jnp.float32),
            pltpu.SemaphoreType.DMA((3, _SPW // 16)),
        ],
    )(_sc_body)

    flat = run(predT, targT)
    partials = flat.reshape(_NW, 128)[:, :16]
    loss = jnp.sum(partials)
    return (loss, {"a0_loss": loss})
